# baseline (speedup 1.0000x reference)
import functools

import jax
import jax.numpy as jnp
from jax import lax
from jax.experimental import pallas as pl
from jax.experimental.pallas import tpu as pltpu
from jax.experimental.pallas import tpu_sc as plsc

N = 10000
E = 320000
D = 128
NCORE = 2
NSUB = 16
NW = NCORE * NSUB
K = 80
NROWS = E // K
GSZ = 8
NG = NROWS // GSZ
NGW = (NG + NW - 1) // NW
RPT = 632
RPT_LAST = N - 15 * RPT

_mesh = plsc.VectorSubcoreMesh(core_axis_name="c", subcore_axis_name="s")


@functools.partial(
    pl.kernel,
    mesh=_mesh,
    out_type=(
        jax.ShapeDtypeStruct((N,), jnp.float32),
        jax.ShapeDtypeStruct((N,), jnp.float32),
    ),
    scratch_types=[
        pltpu.VMEM((GSZ, K), jnp.int32),
        pltpu.VMEM((K,), jnp.float32),
        pltpu.VMEM((N,), jnp.float32),
        pltpu.VMEM_SHARED((N,), jnp.float32),
        pltpu.SemaphoreType.DMA,
    ],
)
def _deg_count(dst_hbm, cnt0_hbm, cnt1_hbm, dstv, onesv, zerov, acc, sem):
    cid = lax.axis_index("c")
    sid = lax.axis_index("s")
    wid = cid * NSUB + sid

    def _ones_body(i, _):
        onesv[pl.ds(i * 16, 16)] = jnp.ones((16,), jnp.float32)
        return 0
    lax.fori_loop(0, K // 16, _ones_body, 0)

    @pl.when(sid == 0)
    def _():
        def _zb(i, _):
            zerov[pl.ds(i * 16, 16)] = jnp.zeros((16,), jnp.float32)
            return 0
        lax.fori_loop(0, N // 16, _zb, 0)
        pltpu.sync_copy(zerov, acc)

    plsc.subcore_barrier()

    def _group(t, _):
        gidx = t * NW + wid

        @pl.when(gidx < NG)
        def _():
            pltpu.sync_copy(dst_hbm.at[gidx], dstv)
            for j in range(GSZ):
                pltpu.async_copy(onesv, acc.at[dstv.at[j]], sem, add=True)
            for j in range(GSZ):
                pltpu.make_async_copy(onesv, acc.at[dstv.at[j]], sem).wait()
        return 0
    lax.fori_loop(0, NGW, _group, 0)

    plsc.subcore_barrier()

    @pl.when((sid == 0) & (cid == 0))
    def _():
        pltpu.sync_copy(acc, cnt0_hbm)

    @pl.when((sid == 0) & (cid == 1))
    def _():
        pltpu.sync_copy(acc, cnt1_hbm)


_NRB = 3
_NIB = 4
_NCHW = NGW * GSZ


@functools.partial(
    pl.kernel,
    mesh=_mesh,
    out_type=(
        jax.ShapeDtypeStruct((N, D), jnp.float32),
        jax.ShapeDtypeStruct((N, D), jnp.float32),
    ),
    scratch_types=(
        [pltpu.VMEM((GSZ, K), jnp.int32) for _ in range(2 * _NIB)]
        + [pltpu.VMEM((K, D), jnp.float32) for _ in range(_NRB)]
        + [pltpu.VMEM_SHARED((N, D), jnp.float32)]
        + [pltpu.SemaphoreType.DMA for _ in range(_NIB + 2 * _NRB)]
    ),
)
def _edge_agg(g_hbm, src_hbm, dst_hbm, p0_hbm, p1_hbm, *refs):
    srcvs = refs[0:_NIB]
    dstvs = refs[_NIB:2 * _NIB]
    rows = refs[2 * _NIB:2 * _NIB + _NRB]
    acc = refs[2 * _NIB + _NRB]
    _s0 = 2 * _NIB + _NRB + 1
    isems = refs[_s0:_s0 + _NIB]
    gsems = refs[_s0 + _NIB:_s0 + _NIB + _NRB]
    ssems = refs[_s0 + _NIB + _NRB:_s0 + _NIB + 2 * _NRB]

    cid = lax.axis_index("c")
    sid = lax.axis_index("s")
    wid = cid * NSUB + sid

    def _zb(i, _):
        for cc in range(D // 16):
            rows[0][i, pl.ds(cc * 16, 16)] = jnp.zeros((16,), jnp.float32)
        return 0
    lax.fori_loop(0, K, _zb, 0)

    @pl.when(sid < 15)
    def _():
        def _zcpy(j, _):
            pltpu.sync_copy(rows[0], acc.at[pl.ds(sid * RPT + j * K, K)])
            return 0
        lax.fori_loop(0, RPT // K, _zcpy, 0)
        pltpu.sync_copy(rows[0].at[pl.ds(0, RPT - (RPT // K) * K)],
                        acc.at[pl.ds(sid * RPT + (RPT // K) * K,
                                     RPT - (RPT // K) * K)])

    @pl.when(sid == 15)
    def _():
        def _zcpy(j, _):
            pltpu.sync_copy(rows[0], acc.at[pl.ds(15 * RPT + j * K, K)])
            return 0
        lax.fori_loop(0, RPT_LAST // K, _zcpy, 0)
        pltpu.sync_copy(rows[0].at[pl.ds(0, RPT_LAST - (RPT_LAST // K) * K)],
                        acc.at[pl.ds(15 * RPT + (RPT_LAST // K) * K,
                                     RPT_LAST - (RPT_LAST // K) * K)])

    plsc.subcore_barrier()

    ntail = _NCHW - GSZ
    tail_w = NG - (NGW - 1) * NW

    def _guarded(need_guard, fn):
        if need_guard:
            pl.when(wid < tail_w)(fn)
        else:
            fn()

    def _issue_idx(t):
        b = t % _NIB
        gidx = t * NW + wid
        pltpu.async_copy(src_hbm.at[gidx], srcvs[b], isems[b])
        pltpu.async_copy(dst_hbm.at[gidx], dstvs[b], isems[b])

    def _wait_idx(t):
        b = t % _NIB
        gidx = t * NW + wid
        pltpu.make_async_copy(src_hbm.at[gidx], srcvs[b], isems[b]).wait()
        pltpu.make_async_copy(dst_hbm.at[gidx], dstvs[b], isems[b]).wait()

    def _issue_gather(c):
        t, j = divmod(c, GSZ)
        pltpu.async_copy(g_hbm.at[srcvs[t % _NIB].at[j]], rows[c % _NRB],
                         gsems[c % _NRB])

    def _wait_gather(c):
        t, j = divmod(c, GSZ)
        pltpu.make_async_copy(g_hbm.at[srcvs[t % _NIB].at[j]], rows[c % _NRB],
                              gsems[c % _NRB]).wait()

    def _issue_scatter(c):
        t, j = divmod(c, GSZ)
        pltpu.async_copy(rows[c % _NRB], acc.at[dstvs[t % _NIB].at[j]],
                         ssems[c % _NRB], add=True)

    def _wait_scatter(c):
        t, j = divmod(c, GSZ)
        pltpu.make_async_copy(rows[c % _NRB], acc.at[dstvs[t % _NIB].at[j]],
                              ssems[c % _NRB]).wait()

    _issue_idx(0)
    _wait_idx(0)
    _issue_idx(1)
    _issue_idx(2)
    _issue_gather(0)

    for c in range(_NCHW):
        t, j = divmod(c, GSZ)
        if j == 2 and t + 3 < NGW:
            _guarded(t + 3 == NGW - 1, lambda tt=t + 3: _issue_idx(tt))
        if j == 5 and t + 1 < NGW:
            _guarded(t + 1 == NGW - 1, lambda tt=t + 1: _wait_idx(tt))
        if c >= 2:
            _guarded(c - 2 >= ntail, lambda cc=c - 2: _wait_scatter(cc))
        if c + 1 < _NCHW:
            _guarded(c + 1 >= ntail, lambda cc=c + 1: _issue_gather(cc))

        def _work(cc=c):
            _wait_gather(cc)
            _issue_scatter(cc)
        _guarded(c >= ntail, _work)

    def _drain():
        _wait_scatter(_NCHW - 2)
        _wait_scatter(_NCHW - 1)
    _guarded(True, _drain)

    plsc.subcore_barrier()

    sl_full = pl.ds(sid * RPT, RPT)
    sl_last = pl.ds(15 * RPT, RPT_LAST)

    @pl.when((cid == 0) & (sid < 15))
    def _():
        pltpu.sync_copy(acc.at[sl_full], p0_hbm.at[sl_full])

    @pl.when((cid == 0) & (sid == 15))
    def _():
        pltpu.sync_copy(acc.at[sl_last], p0_hbm.at[sl_last])

    @pl.when((cid == 1) & (sid < 15))
    def _():
        pltpu.sync_copy(acc.at[sl_full], p1_hbm.at[sl_full])

    @pl.when((cid == 1) & (sid == 15))
    def _():
        pltpu.sync_copy(acc.at[sl_last], p1_hbm.at[sl_last])


_RB = 1000


def _pre1_body(x_ref, w_ref, c0_ref, c1_ref, g_ref, dis_ref):
    deg = c0_ref[...] + c1_ref[...] + 1.0
    dis = lax.rsqrt(deg)
    y = jnp.dot(x_ref[...], w_ref[...], preferred_element_type=jnp.float32)
    g = y * dis
    g_ref[...] = g
    dis_ref[...] = jnp.broadcast_to(dis, (g.shape[0], D))


def _pre1(x, w1, c0, c1):
    return pl.pallas_call(
        _pre1_body,
        grid=(N // _RB,),
        in_specs=[
            pl.BlockSpec((_RB, D), lambda i: (i, 0)),
            pl.BlockSpec((D, D), lambda i: (0, 0)),
            pl.BlockSpec((_RB, 1), lambda i: (i, 0)),
            pl.BlockSpec((_RB, 1), lambda i: (i, 0)),
        ],
        out_specs=[
            pl.BlockSpec((_RB, D), lambda i: (i, 0)),
            pl.BlockSpec((_RB, D), lambda i: (i, 0)),
        ],
        out_shape=[
            jax.ShapeDtypeStruct((N, D), jnp.float32),
            jax.ShapeDtypeStruct((N, D), jnp.float32),
        ],
    )(x, w1, c0, c1)


def _mid_body(p0_ref, p1_ref, gp_ref, dis_ref, b_ref, w_ref, gn_ref):
    dis = dis_ref[...]
    h = jnp.maximum((p0_ref[...] + p1_ref[...] + gp_ref[...]) * dis
                    + b_ref[...], 0.0)
    gn_ref[...] = jnp.dot(h, w_ref[...],
                          preferred_element_type=jnp.float32) * dis


def _mid(p0, p1, gp, dis, b, w):
    return pl.pallas_call(
        _mid_body,
        grid=(N // _RB,),
        in_specs=[
            pl.BlockSpec((_RB, D), lambda i: (i, 0)),
            pl.BlockSpec((_RB, D), lambda i: (i, 0)),
            pl.BlockSpec((_RB, D), lambda i: (i, 0)),
            pl.BlockSpec((_RB, D), lambda i: (i, 0)),
            pl.BlockSpec((1, D), lambda i: (0, 0)),
            pl.BlockSpec((D, D), lambda i: (0, 0)),
        ],
        out_specs=pl.BlockSpec((_RB, D), lambda i: (i, 0)),
        out_shape=jax.ShapeDtypeStruct((N, D), jnp.float32),
    )(p0, p1, gp, dis, b, w)


def _fin_body(p0_ref, p1_ref, gp_ref, dis_ref, b_ref, wl_ref, bl_ref, out_ref):
    h = jnp.maximum((p0_ref[...] + p1_ref[...] + gp_ref[...]) * dis_ref[...]
                    + b_ref[...], 0.0)
    out_ref[...] = jnp.dot(h, wl_ref[...],
                           preferred_element_type=jnp.float32) + bl_ref[...]


def _fin(p0, p1, gp, dis, b, wl_pad, bl_pad):
    return pl.pallas_call(
        _fin_body,
        grid=(1,),
        in_specs=[
            pl.BlockSpec((8, D), lambda i: (0, 0)),
            pl.BlockSpec((8, D), lambda i: (0, 0)),
            pl.BlockSpec((8, D), lambda i: (0, 0)),
            pl.BlockSpec((8, D), lambda i: (0, 0)),
            pl.BlockSpec((1, D), lambda i: (0, 0)),
            pl.BlockSpec((D, D), lambda i: (0, 0)),
            pl.BlockSpec((1, D), lambda i: (0, 0)),
        ],
        out_specs=pl.BlockSpec((8, D), lambda i: (0, 0)),
        out_shape=jax.ShapeDtypeStruct((8, D), jnp.float32),
    )(p0, p1, gp, dis, b, wl_pad, bl_pad)


def kernel(x, edge_index, W1, b1, W2, b2, W3, b3, Wl, bl):
    src2 = edge_index[0].reshape(NG, GSZ, K)
    dst2 = edge_index[1].reshape(NG, GSZ, K)

    cnt0, cnt1 = _deg_count(dst2)
    c0 = cnt0[:, None]
    c1 = cnt1[:, None]

    g1, dis = _pre1(x, W1, c0, c1)
    p0, p1 = _edge_agg(g1, src2, dst2)
    g2 = _mid(p0, p1, g1, dis, b1[None, :], W2)
    p0, p1 = _edge_agg(g2, src2, dst2)
    g3 = _mid(p0, p1, g2, dis, b2[None, :], W3)
    p0, p1 = _edge_agg(g3, src2, dst2)

    wl_pad = jnp.pad(Wl, ((0, 0), (0, D - Wl.shape[1])))
    bl_pad = jnp.pad(bl, (0, D - bl.shape[0]))[None, :]
    out = _fin(p0, p1, g3, dis, b3[None, :], wl_pad, bl_pad)
    return out[:5, :3]
